# baseline (speedup 1.0000x reference)
import functools

import jax
import jax.numpy as jnp
from jax import lax
from jax.experimental import pallas as pl
from jax.experimental.pallas import tpu as pltpu
from jax.experimental.pallas import tpu_sc as plsc

B, C, H, W = 8, 3, 512, 512
HW = H * W
NW = 32
SHARD = HW // NW
NPC = 64
PCPTS = (B * HW) // NPC
PCB = NPC // B
ACH = 2048
CAP = 256
NBKT = 512
BINSZ = NPC * NBKT * CAP
DUMP = BINSZ
ROWS = 128


def _prep_body(flow_ref, depth_ref, lin_ref, pvn_ref):
    r = pl.program_id(1)
    fx = flow_ref[0, 0]
    fy = flow_ref[0, 1]
    gx = lax.broadcasted_iota(jnp.int32, (ROWS, W), 1).astype(jnp.float32)
    gy = lax.broadcasted_iota(jnp.int32, (ROWS, W), 0).astype(jnp.float32) \
        + (r * ROWS).astype(jnp.float32)
    cxf = jnp.round(gx + fx)
    cyf = jnp.round(gy + fy)
    inb = (cxf >= 0) & (cxf < W) & (cyf >= 0) & (cyf < H)
    cx = jnp.clip(cxf, 0, W - 1).astype(jnp.int32)
    cy = jnp.clip(cyf, 0, H - 1).astype(jnp.int32)
    lin_ref[0] = jnp.where(inb, cy * W + cx, 0)
    v = depth_ref[0, 0]
    pvn = 1.0 / (v + 1e-08)
    pvn_ref[0] = pvn * (pvn < 10000.0).astype(jnp.float32)


def _prep(x, flow_in):
    lin, pvn = pl.pallas_call(
        _prep_body,
        out_shape=(
            jax.ShapeDtypeStruct((B, H, W), jnp.int32),
            jax.ShapeDtypeStruct((B, H, W), jnp.float32),
        ),
        grid=(B, H // ROWS),
        in_specs=[
            pl.BlockSpec((1, 2, ROWS, W), lambda b, r: (b, 0, r, 0)),
            pl.BlockSpec((1, 1, ROWS, W), lambda b, r: (b, 2, r, 0)),
        ],
        out_specs=(
            pl.BlockSpec((1, ROWS, W), lambda b, r: (b, r, 0)),
            pl.BlockSpec((1, ROWS, W), lambda b, r: (b, r, 0)),
        ),
    )(flow_in, x)
    return lin.reshape(B * HW), pvn.reshape(B * HW)


def _bin_body(lin_hbm, pvn_hbm, pack_hbm, pvnb_hbm, cnt_hbm,
              lin_v, pvn_v, next_v, pstage, vstage, posb, sem):
    wid = lax.axis_index("s") * 2 + lax.axis_index("c")
    iota = lax.iota(jnp.int32, 16)

    def per_pc(k, _):
        pc = wid * 2 + k
        pt0 = pc * PCPTS
        p_base = (pc % PCB) * PCPTS

        def zero(i, _):
            next_v[pl.ds(i * 16, 16)] = jnp.zeros((16,), jnp.int32)
            return 0

        lax.fori_loop(0, 1024 // 16, zero, 0)

        def per_chunk(ch, _):
            off = pt0 + ch * ACH
            pltpu.sync_copy(lin_hbm.at[pl.ds(off, ACH)], lin_v)
            pltpu.sync_copy(pvn_hbm.at[pl.ds(off, ACH)], pvn_v)

            def vloop(i, _):
                l = lin_v[pl.ds(i * 16, 16)]
                v = pvn_v[pl.ds(i * 16, 16)]
                act = v > 0.0
                rk = lax.shift_right_logical(l, 9)
                key = jnp.where(act, rk, 1023)
                p = p_base + ch * ACH + i * 16 + iota
                pack = lax.shift_left(l & 511, 18) | p
                skey, sval = plsc.sort_key_val(key, iota)
                v_s = jnp.take(v, sval)
                pack_s = jnp.take(pack, sval)
                act_s = skey < NBKT
                st = iota
                c = ((skey == jnp.take(skey, jnp.maximum(iota - 1, 0)))
                     & (iota >= 1)).astype(jnp.int32)
                for d in (1, 2, 4, 8):
                    back = jnp.maximum(iota - d, 0)
                    st = jnp.where(c != 0, jnp.take(st, back), st)
                    c = c & jnp.take(c, back)
                rank = iota - st
                nxt_key = jnp.take(skey, jnp.minimum(iota + 1, 15))
                is_last = (iota == 15) | (nxt_key != skey)
                cur = plsc.load_gather(next_v, [skey])
                pos = cur + rank
                valid = act_s & (pos < CAP)
                plsc.store_scatter(next_v, [skey], pos + 1, mask=is_last & act_s)
                gpos = (pc * NBKT + skey) * CAP + pos
                dump = DUMP + wid * ACH + i * 16 + iota
                posb[pl.ds(i * 16, 16)] = jnp.where(valid, gpos, dump)
                pstage[pl.ds(i * 16, 16)] = pack_s
                vstage[pl.ds(i * 16, 16)] = v_s
                return 0

            lax.fori_loop(0, ACH // 16, vloop, 0)
            pltpu.async_copy(pstage, pack_hbm.at[posb], sem).wait()
            pltpu.async_copy(vstage, pvnb_hbm.at[posb], sem).wait()
            return 0

        lax.fori_loop(0, PCPTS // ACH, per_chunk, 0)
        pltpu.sync_copy(next_v.at[pl.ds(0, NBKT)],
                        cnt_hbm.at[pl.ds(pc * NBKT, NBKT)])
        return 0

    lax.fori_loop(0, 2, per_pc, 0)


def _splat_body(pack_hbm, pvnb_hbm, cnt_hbm, x_hbm, lin_hbm, pvn_hbm, out_hbm,
                pkbuf, pvbuf, cntv, maxv_fb, argp_fb, idx_v, gath_v, outb_v,
                lin_v, pvn_v, sem):
    wid = lax.axis_index("s") * 2 + lax.axis_index("c")
    base = wid * SHARD
    iota = lax.iota(jnp.int32, 16)

    def per_batch(b, _):
        pt_base = b * HW

        def init(i, _):
            maxv_fb[pl.ds(i * 16, 16)] = jnp.zeros((16,), jnp.float32)
            argp_fb[pl.ds(i * 16, 16)] = jnp.full((16,), HW, jnp.int32)
            return 0

        lax.fori_loop(0, SHARD // 16, init, 0)

        handles = []
        for j in range(PCB):
            pc = b * PCB + j
            boff = (pc * NBKT + wid * 16) * CAP
            handles.append(pltpu.async_copy(
                cnt_hbm.at[pl.ds(pc * NBKT + wid * 16, 16)],
                cntv.at[pl.ds(j * 16, 16)], sem))
            handles.append(pltpu.async_copy(
                pack_hbm.at[pl.ds(boff, 16 * CAP)],
                pkbuf.at[pl.ds(j * 16 * CAP, 16 * CAP)], sem))
            handles.append(pltpu.async_copy(
                pvnb_hbm.at[pl.ds(boff, 16 * CAP)],
                pvbuf.at[pl.ds(j * 16 * CAP, 16 * CAP)], sem))
        for h in handles:
            h.wait()

        def ovf_scan(j, m):
            cj = cntv[pl.ds(j * 16, 16)]
            return jnp.maximum(m, jnp.max(cj))

        max_cnt = lax.fori_loop(0, PCB, ovf_scan, jnp.int32(0))

        def pass1_j(j, _):
            cj = jnp.minimum(cntv[pl.ds(j * 16, 16)], CAP)
            trip = jnp.max(cj)
            bj = j * 16 * CAP

            def it(i, _):
                vidx = bj + iota * CAP + i
                pk = plsc.load_gather(pkbuf, [vidx])
                v = plsc.load_gather(pvbuf, [vidx])
                actm = i < cj
                low9 = lax.shift_right_logical(pk, 18)
                floc = iota * 512 + low9
                g = plsc.load_gather(maxv_fb, [floc])
                m = actm & (v > g)
                plsc.store_scatter(maxv_fb, [floc], v, mask=m)
                return 0

            lax.fori_loop(0, trip, it, 0)
            return 0

        lax.fori_loop(0, PCB, pass1_j, 0)

        @pl.when(max_cnt > CAP)
        def _():
            def f1_chunk(ci, _):
                off = pt_base + ci * ACH
                pltpu.sync_copy(lin_hbm.at[pl.ds(off, ACH)], lin_v)
                pltpu.sync_copy(pvn_hbm.at[pl.ds(off, ACH)], pvn_v)

                def vloop(i, _):
                    l = lin_v[pl.ds(i * 16, 16)]
                    v = pvn_v[pl.ds(i * 16, 16)]
                    loc = l - base
                    act = (loc >= 0) & (loc < SHARD) & (v > 0.0)
                    locs = jnp.clip(loc, 0, SHARD - 1)
                    g = plsc.load_gather(maxv_fb, [locs])
                    need = act & (v > g)

                    def body(m):
                        plsc.store_scatter(maxv_fb, [locs], v, mask=m != 0)
                        g2 = plsc.load_gather(maxv_fb, [locs])
                        return (act & (v > g2)).astype(jnp.int32)

                    lax.while_loop(lambda m: jnp.any(m != 0), body,
                                   need.astype(jnp.int32))
                    return 0

                lax.fori_loop(0, ACH // 16, vloop, 0)
                return 0

            lax.fori_loop(0, HW // ACH, f1_chunk, 0)

        def pass2_j(j, _):
            cj = jnp.minimum(cntv[pl.ds(j * 16, 16)], CAP)
            trip = jnp.max(cj)
            bj = j * 16 * CAP

            def it(i, _):
                vidx = bj + iota * CAP + i
                pk = plsc.load_gather(pkbuf, [vidx])
                v = plsc.load_gather(pvbuf, [vidx])
                actm = i < cj
                low9 = lax.shift_right_logical(pk, 18)
                p = pk & 0x3FFFF
                floc = iota * 512 + low9
                g = plsc.load_gather(maxv_fb, [floc])
                win = actm & (v == g)
                ga = plsc.load_gather(argp_fb, [floc])
                m = win & (p < ga)
                plsc.store_scatter(argp_fb, [floc], p, mask=m)
                return 0

            lax.fori_loop(0, trip, it, 0)
            return 0

        lax.fori_loop(0, PCB, pass2_j, 0)

        @pl.when(max_cnt > CAP)
        def _():
            def f2_chunk(ci, _):
                off = pt_base + ci * ACH
                pltpu.sync_copy(lin_hbm.at[pl.ds(off, ACH)], lin_v)
                pltpu.sync_copy(pvn_hbm.at[pl.ds(off, ACH)], pvn_v)

                def vloop(i, _):
                    l = lin_v[pl.ds(i * 16, 16)]
                    v = pvn_v[pl.ds(i * 16, 16)]
                    loc = l - base
                    act = (loc >= 0) & (loc < SHARD) & (v > 0.0)
                    locs = jnp.clip(loc, 0, SHARD - 1)
                    p = ci * ACH + i * 16 + iota
                    g = plsc.load_gather(maxv_fb, [locs])
                    win = act & (v == g)
                    ga = plsc.load_gather(argp_fb, [locs])
                    need = win & (p < ga)

                    def body(m):
                        plsc.store_scatter(argp_fb, [locs], p, mask=m != 0)
                        ga2 = plsc.load_gather(argp_fb, [locs])
                        return (win & (p < ga2)).astype(jnp.int32)

                    lax.while_loop(lambda m: jnp.any(m != 0), body,
                                   need.astype(jnp.int32))
                    return 0

                lax.fori_loop(0, ACH // 16, vloop, 0)
                return 0

            lax.fori_loop(0, HW // ACH, f2_chunk, 0)

        def mkidx(i, _):
            q = base + i * 16 + iota
            a = argp_fb[pl.ds(i * 16, 16)]
            valid = (a < HW) & (q > 0)
            idx_v[pl.ds(i * 16, 16)] = jnp.where(valid, a, q) + (b * C) * HW
            return 0

        lax.fori_loop(0, SHARD // 16, mkidx, 0)

        for c in range(C):
            if c > 0:
                def bump(i, _):
                    idx_v[pl.ds(i * 16, 16)] = idx_v[pl.ds(i * 16, 16)] + HW
                    return 0

                lax.fori_loop(0, SHARD // 16, bump, 0)
            pltpu.async_copy(x_hbm.at[idx_v], gath_v, sem).wait()

            def emit(i, _):
                q = base + i * 16 + iota
                a = argp_fb[pl.ds(i * 16, 16)]
                valid = (a < HW) & (q > 0)
                gv = gath_v[pl.ds(i * 16, 16)]
                outb_v[pl.ds(i * 16, 16)] = jnp.where(
                    valid & (gv < 10000.0), gv, 0.0)
                return 0

            lax.fori_loop(0, SHARD // 16, emit, 0)
            pltpu.sync_copy(outb_v, out_hbm.at[pl.ds((b * C + c) * HW + base, SHARD)])
        return 0

    lax.fori_loop(0, B, per_batch, 0)


@jax.jit
def kernel(x, flow_in):
    lin, pvn = _prep(x, flow_in)
    xf = x.reshape(B * C * HW)
    mesh = plsc.VectorSubcoreMesh(core_axis_name="c", subcore_axis_name="s")
    bink = functools.partial(
        pl.kernel,
        mesh=mesh,
        compiler_params=pltpu.CompilerParams(needs_layout_passes=False),
        out_type=(
            jax.ShapeDtypeStruct((BINSZ + NW * ACH,), jnp.int32),
            jax.ShapeDtypeStruct((BINSZ + NW * ACH,), jnp.float32),
            jax.ShapeDtypeStruct((NPC * NBKT,), jnp.int32),
        ),
        scratch_types=[
            pltpu.VMEM((ACH,), jnp.int32),
            pltpu.VMEM((ACH,), jnp.float32),
            pltpu.VMEM((1024,), jnp.int32),
            pltpu.VMEM((ACH,), jnp.int32),
            pltpu.VMEM((ACH,), jnp.float32),
            pltpu.VMEM((ACH,), jnp.int32),
            pltpu.SemaphoreType.DMA,
        ],
    )(_bin_body)
    pack_b, pvn_b, counts = bink(lin, pvn)

    splat = functools.partial(
        pl.kernel,
        mesh=mesh,
        compiler_params=pltpu.CompilerParams(needs_layout_passes=False),
        out_type=jax.ShapeDtypeStruct((B * C * HW,), jnp.float32),
        scratch_types=[
            pltpu.VMEM((PCB * 16 * CAP,), jnp.int32),
            pltpu.VMEM((PCB * 16 * CAP,), jnp.float32),
            pltpu.VMEM((PCB * 16,), jnp.int32),
            pltpu.VMEM((SHARD,), jnp.float32),
            pltpu.VMEM((SHARD,), jnp.int32),
            pltpu.VMEM((SHARD,), jnp.int32),
            pltpu.VMEM((SHARD,), jnp.float32),
            pltpu.VMEM((SHARD,), jnp.float32),
            pltpu.VMEM((ACH,), jnp.int32),
            pltpu.VMEM((ACH,), jnp.float32),
            pltpu.SemaphoreType.DMA,
        ],
    )(_splat_body)
    out = splat(pack_b, pvn_b, counts, xf, lin, pvn)
    return out.reshape(B, C, H, W)


# speedup vs baseline: 1.0242x; 1.0242x over previous
import functools

import jax
import jax.numpy as jnp
from jax import lax
from jax.experimental import pallas as pl
from jax.experimental.pallas import tpu as pltpu
from jax.experimental.pallas import tpu_sc as plsc

B, C, H, W = 8, 3, 512, 512
HW = H * W
NW = 32
SHARD = HW // NW
NPC = 64
PCPTS = (B * HW) // NPC
PCB = NPC // B
ACH = 2048
CAP = 256
NBKT = 512
BINSZ = NPC * NBKT * CAP
DUMP = BINSZ
ROWS = 128


def _prep_body(flow_ref, depth_ref, lin_ref, pvn_ref):
    r = pl.program_id(1)
    fx = flow_ref[0, 0]
    fy = flow_ref[0, 1]
    gx = lax.broadcasted_iota(jnp.int32, (ROWS, W), 1).astype(jnp.float32)
    gy = lax.broadcasted_iota(jnp.int32, (ROWS, W), 0).astype(jnp.float32) \
        + (r * ROWS).astype(jnp.float32)
    cxf = jnp.round(gx + fx)
    cyf = jnp.round(gy + fy)
    inb = (cxf >= 0) & (cxf < W) & (cyf >= 0) & (cyf < H)
    cx = jnp.clip(cxf, 0, W - 1).astype(jnp.int32)
    cy = jnp.clip(cyf, 0, H - 1).astype(jnp.int32)
    lin_ref[0] = jnp.where(inb, cy * W + cx, 0)
    v = depth_ref[0, 0]
    pvn = 1.0 / (v + 1e-08)
    pvn_ref[0] = pvn * (pvn < 10000.0).astype(jnp.float32)


def _prep(x, flow_in):
    lin, pvn = pl.pallas_call(
        _prep_body,
        out_shape=(
            jax.ShapeDtypeStruct((B, H, W), jnp.int32),
            jax.ShapeDtypeStruct((B, H, W), jnp.float32),
        ),
        grid=(B, H // ROWS),
        in_specs=[
            pl.BlockSpec((1, 2, ROWS, W), lambda b, r: (b, 0, r, 0)),
            pl.BlockSpec((1, 1, ROWS, W), lambda b, r: (b, 2, r, 0)),
        ],
        out_specs=(
            pl.BlockSpec((1, ROWS, W), lambda b, r: (b, r, 0)),
            pl.BlockSpec((1, ROWS, W), lambda b, r: (b, r, 0)),
        ),
    )(flow_in, x)
    return lin.reshape(B * HW), pvn.reshape(B * HW)


def _bin_body(lin_hbm, pvn_hbm, pack_hbm, pvnb_hbm, cnt_hbm,
              lin_v, pvn_v, next_v, pstage, vstage, posb, sem):
    wid = lax.axis_index("s") * 2 + lax.axis_index("c")
    iota = lax.iota(jnp.int32, 16)

    def per_pc(k, _):
        pc = wid * 2 + k
        pt0 = pc * PCPTS
        p_base = (pc % PCB) * PCPTS

        def zero(i, _):
            next_v[pl.ds(i * 16, 16)] = jnp.zeros((16,), jnp.int32)
            return 0

        lax.fori_loop(0, 1024 // 16, zero, 0)

        def per_chunk(ch, _):
            off = pt0 + ch * ACH
            pltpu.sync_copy(lin_hbm.at[pl.ds(off, ACH)], lin_v)
            pltpu.sync_copy(pvn_hbm.at[pl.ds(off, ACH)], pvn_v)

            def vloop(i, _):
                l = lin_v[pl.ds(i * 16, 16)]
                v = pvn_v[pl.ds(i * 16, 16)]
                act = v > 0.0
                rk = lax.shift_right_logical(l, 9)
                key = jnp.where(act, rk, 1023)
                p = p_base + ch * ACH + i * 16 + iota
                pack = lax.shift_left(l & 511, 18) | p
                skey, sval = plsc.sort_key_val(key, iota)
                v_s = jnp.take(v, sval)
                pack_s = jnp.take(pack, sval)
                act_s = skey < NBKT
                st = iota
                c = ((skey == jnp.take(skey, jnp.maximum(iota - 1, 0)))
                     & (iota >= 1)).astype(jnp.int32)
                for d in (1, 2, 4, 8):
                    back = jnp.maximum(iota - d, 0)
                    st = jnp.where(c != 0, jnp.take(st, back), st)
                    c = c & jnp.take(c, back)
                rank = iota - st
                nxt_key = jnp.take(skey, jnp.minimum(iota + 1, 15))
                is_last = (iota == 15) | (nxt_key != skey)
                cur = plsc.load_gather(next_v, [skey])
                pos = cur + rank
                valid = act_s & (pos < CAP)
                plsc.store_scatter(next_v, [skey], pos + 1, mask=is_last & act_s)
                gpos = (pc * NBKT + skey) * CAP + pos
                dump = DUMP + pt0 + ch * ACH + i * 16 + iota
                posb[pl.ds(i * 16, 16)] = jnp.where(valid, gpos, dump)
                pstage[pl.ds(i * 16, 16)] = pack_s
                vstage[pl.ds(i * 16, 16)] = v_s
                return 0

            lax.fori_loop(0, ACH // 16, vloop, 0)
            pltpu.async_copy(pstage, pack_hbm.at[posb], sem).wait()
            pltpu.async_copy(vstage, pvnb_hbm.at[posb], sem).wait()
            return 0

        lax.fori_loop(0, PCPTS // ACH, per_chunk, 0)
        pltpu.sync_copy(next_v.at[pl.ds(0, NBKT)],
                        cnt_hbm.at[pl.ds(pc * NBKT, NBKT)])
        return 0

    lax.fori_loop(0, 2, per_pc, 0)


def _splat_body(pack_hbm, pvnb_hbm, cnt_hbm, x_hbm, lin_hbm, pvn_hbm, out_hbm,
                pkbuf, pvbuf, cntv, maxv_fb, argp_fb, idx_v, gath_v, outb_v,
                lin_v, pvn_v, sem):
    wid = lax.axis_index("s") * 2 + lax.axis_index("c")
    base = wid * SHARD
    iota = lax.iota(jnp.int32, 16)

    def per_batch(b, _):
        pt_base = b * HW

        def init(i, _):
            maxv_fb[pl.ds(i * 16, 16)] = jnp.zeros((16,), jnp.float32)
            argp_fb[pl.ds(i * 16, 16)] = jnp.full((16,), HW, jnp.int32)
            return 0

        lax.fori_loop(0, SHARD // 16, init, 0)

        handles = []
        for j in range(PCB):
            pc = b * PCB + j
            boff = (pc * NBKT + wid * 16) * CAP
            handles.append(pltpu.async_copy(
                cnt_hbm.at[pl.ds(pc * NBKT + wid * 16, 16)],
                cntv.at[pl.ds(j * 16, 16)], sem))
            handles.append(pltpu.async_copy(
                pack_hbm.at[pl.ds(boff, 16 * CAP)],
                pkbuf.at[pl.ds(j * 16 * CAP, 16 * CAP)], sem))
            handles.append(pltpu.async_copy(
                pvnb_hbm.at[pl.ds(boff, 16 * CAP)],
                pvbuf.at[pl.ds(j * 16 * CAP, 16 * CAP)], sem))
        for h in handles:
            h.wait()

        def ovf_scan(j, m):
            cj = cntv[pl.ds(j * 16, 16)]
            return jnp.maximum(m, jnp.max(cj))

        max_cnt = lax.fori_loop(0, PCB, ovf_scan, jnp.int32(0))

        def pass1_j(j, _):
            cj = jnp.minimum(cntv[pl.ds(j * 16, 16)], CAP)
            trip = jnp.max(cj)
            bj = j * 16 * CAP

            def it(i, _):
                vidx = bj + iota * CAP + i
                pk = plsc.load_gather(pkbuf, [vidx])
                v = plsc.load_gather(pvbuf, [vidx])
                actm = i < cj
                low9 = lax.shift_right_logical(pk, 18)
                floc = iota * 512 + low9
                g = plsc.load_gather(maxv_fb, [floc])
                m = actm & (v > g)
                plsc.store_scatter(maxv_fb, [floc], v, mask=m)
                return 0

            lax.fori_loop(0, trip, it, 0)
            return 0

        lax.fori_loop(0, PCB, pass1_j, 0)

        @pl.when(max_cnt > CAP)
        def _():
            def f1_chunk(ci, _):
                off = pt_base + ci * ACH
                pltpu.sync_copy(lin_hbm.at[pl.ds(off, ACH)], lin_v)
                pltpu.sync_copy(pvn_hbm.at[pl.ds(off, ACH)], pvn_v)

                def vloop(i, _):
                    l = lin_v[pl.ds(i * 16, 16)]
                    v = pvn_v[pl.ds(i * 16, 16)]
                    loc = l - base
                    act = (loc >= 0) & (loc < SHARD) & (v > 0.0)
                    locs = jnp.clip(loc, 0, SHARD - 1)
                    g = plsc.load_gather(maxv_fb, [locs])
                    need = act & (v > g)

                    def body(m):
                        plsc.store_scatter(maxv_fb, [locs], v, mask=m != 0)
                        g2 = plsc.load_gather(maxv_fb, [locs])
                        return (act & (v > g2)).astype(jnp.int32)

                    lax.while_loop(lambda m: jnp.any(m != 0), body,
                                   need.astype(jnp.int32))
                    return 0

                lax.fori_loop(0, ACH // 16, vloop, 0)
                return 0

            lax.fori_loop(0, HW // ACH, f1_chunk, 0)

        def pass2_j(j, _):
            cj = jnp.minimum(cntv[pl.ds(j * 16, 16)], CAP)
            trip = jnp.max(cj)
            bj = j * 16 * CAP

            def it(i, _):
                vidx = bj + iota * CAP + i
                pk = plsc.load_gather(pkbuf, [vidx])
                v = plsc.load_gather(pvbuf, [vidx])
                actm = i < cj
                low9 = lax.shift_right_logical(pk, 18)
                p = pk & 0x3FFFF
                floc = iota * 512 + low9
                g = plsc.load_gather(maxv_fb, [floc])
                win = actm & (v == g)
                ga = plsc.load_gather(argp_fb, [floc])
                m = win & (p < ga)
                plsc.store_scatter(argp_fb, [floc], p, mask=m)
                return 0

            lax.fori_loop(0, trip, it, 0)
            return 0

        lax.fori_loop(0, PCB, pass2_j, 0)

        @pl.when(max_cnt > CAP)
        def _():
            def f2_chunk(ci, _):
                off = pt_base + ci * ACH
                pltpu.sync_copy(lin_hbm.at[pl.ds(off, ACH)], lin_v)
                pltpu.sync_copy(pvn_hbm.at[pl.ds(off, ACH)], pvn_v)

                def vloop(i, _):
                    l = lin_v[pl.ds(i * 16, 16)]
                    v = pvn_v[pl.ds(i * 16, 16)]
                    loc = l - base
                    act = (loc >= 0) & (loc < SHARD) & (v > 0.0)
                    locs = jnp.clip(loc, 0, SHARD - 1)
                    p = ci * ACH + i * 16 + iota
                    g = plsc.load_gather(maxv_fb, [locs])
                    win = act & (v == g)
                    ga = plsc.load_gather(argp_fb, [locs])
                    need = win & (p < ga)

                    def body(m):
                        plsc.store_scatter(argp_fb, [locs], p, mask=m != 0)
                        ga2 = plsc.load_gather(argp_fb, [locs])
                        return (win & (p < ga2)).astype(jnp.int32)

                    lax.while_loop(lambda m: jnp.any(m != 0), body,
                                   need.astype(jnp.int32))
                    return 0

                lax.fori_loop(0, ACH // 16, vloop, 0)
                return 0

            lax.fori_loop(0, HW // ACH, f2_chunk, 0)

        def mkidx(i, _):
            q = base + i * 16 + iota
            a = argp_fb[pl.ds(i * 16, 16)]
            valid = (a < HW) & (q > 0)
            idx_v[pl.ds(i * 16, 16)] = jnp.where(valid, a, q) + (b * C) * HW
            return 0

        lax.fori_loop(0, SHARD // 16, mkidx, 0)

        for c in range(C):
            if c > 0:
                def bump(i, _):
                    idx_v[pl.ds(i * 16, 16)] = idx_v[pl.ds(i * 16, 16)] + HW
                    return 0

                lax.fori_loop(0, SHARD // 16, bump, 0)
            pltpu.async_copy(x_hbm.at[idx_v], gath_v, sem).wait()

            def emit(i, _):
                q = base + i * 16 + iota
                a = argp_fb[pl.ds(i * 16, 16)]
                valid = (a < HW) & (q > 0)
                gv = gath_v[pl.ds(i * 16, 16)]
                outb_v[pl.ds(i * 16, 16)] = jnp.where(
                    valid & (gv < 10000.0), gv, 0.0)
                return 0

            lax.fori_loop(0, SHARD // 16, emit, 0)
            pltpu.sync_copy(outb_v, out_hbm.at[pl.ds((b * C + c) * HW + base, SHARD)])
        return 0

    lax.fori_loop(0, B, per_batch, 0)


@jax.jit
def kernel(x, flow_in):
    lin, pvn = _prep(x, flow_in)
    xf = x.reshape(B * C * HW)
    mesh = plsc.VectorSubcoreMesh(core_axis_name="c", subcore_axis_name="s")
    bink = functools.partial(
        pl.kernel,
        mesh=mesh,
        compiler_params=pltpu.CompilerParams(needs_layout_passes=False),
        out_type=(
            jax.ShapeDtypeStruct((BINSZ + B * HW,), jnp.int32),
            jax.ShapeDtypeStruct((BINSZ + B * HW,), jnp.float32),
            jax.ShapeDtypeStruct((NPC * NBKT,), jnp.int32),
        ),
        scratch_types=[
            pltpu.VMEM((ACH,), jnp.int32),
            pltpu.VMEM((ACH,), jnp.float32),
            pltpu.VMEM((1024,), jnp.int32),
            pltpu.VMEM((ACH,), jnp.int32),
            pltpu.VMEM((ACH,), jnp.float32),
            pltpu.VMEM((ACH,), jnp.int32),
            pltpu.SemaphoreType.DMA,
        ],
    )(_bin_body)
    pack_b, pvn_b, counts = bink(lin, pvn)

    splat = functools.partial(
        pl.kernel,
        mesh=mesh,
        compiler_params=pltpu.CompilerParams(needs_layout_passes=False),
        out_type=jax.ShapeDtypeStruct((B * C * HW,), jnp.float32),
        scratch_types=[
            pltpu.VMEM((PCB * 16 * CAP,), jnp.int32),
            pltpu.VMEM((PCB * 16 * CAP,), jnp.float32),
            pltpu.VMEM((PCB * 16,), jnp.int32),
            pltpu.VMEM((SHARD,), jnp.float32),
            pltpu.VMEM((SHARD,), jnp.int32),
            pltpu.VMEM((SHARD,), jnp.int32),
            pltpu.VMEM((SHARD,), jnp.float32),
            pltpu.VMEM((SHARD,), jnp.float32),
            pltpu.VMEM((ACH,), jnp.int32),
            pltpu.VMEM((ACH,), jnp.float32),
            pltpu.SemaphoreType.DMA,
        ],
    )(_splat_body)
    out = splat(pack_b, pvn_b, counts, xf, lin, pvn)
    return out.reshape(B, C, H, W)


# speedup vs baseline: 2.9908x; 2.9202x over previous
import functools

import jax
import jax.numpy as jnp
from jax import lax
from jax.experimental import pallas as pl
from jax.experimental.pallas import tpu as pltpu
from jax.experimental.pallas import tpu_sc as plsc

B, C, H, W = 8, 3, 512, 512
HW = H * W
NW = 32
SHARD = HW // NW
NPC = 64
PCPTS = (B * HW) // NPC
PCB = NPC // B
ACH = 2048
CAP = 256
NBKT = 512
BINSZ = NPC * NBKT * CAP
DUMP = BINSZ
ROWS = 128


def _prep_body(flow_ref, depth_ref, lin_ref, pvn_ref):
    r = pl.program_id(1)
    fx = flow_ref[0, 0]
    fy = flow_ref[0, 1]
    gx = lax.broadcasted_iota(jnp.int32, (ROWS, W), 1).astype(jnp.float32)
    gy = lax.broadcasted_iota(jnp.int32, (ROWS, W), 0).astype(jnp.float32) \
        + (r * ROWS).astype(jnp.float32)
    cxf = jnp.round(gx + fx)
    cyf = jnp.round(gy + fy)
    inb = (cxf >= 0) & (cxf < W) & (cyf >= 0) & (cyf < H)
    cx = jnp.clip(cxf, 0, W - 1).astype(jnp.int32)
    cy = jnp.clip(cyf, 0, H - 1).astype(jnp.int32)
    lin_ref[0] = jnp.where(inb, cy * W + cx, 0)
    v = depth_ref[0, 0]
    pvn = 1.0 / (v + 1e-08)
    pvn_ref[0] = pvn * (pvn < 10000.0).astype(jnp.float32)


def _prep(x, flow_in):
    lin, pvn = pl.pallas_call(
        _prep_body,
        out_shape=(
            jax.ShapeDtypeStruct((B, H, W), jnp.int32),
            jax.ShapeDtypeStruct((B, H, W), jnp.float32),
        ),
        grid=(B, H // ROWS),
        in_specs=[
            pl.BlockSpec((1, 2, ROWS, W), lambda b, r: (b, 0, r, 0)),
            pl.BlockSpec((1, 1, ROWS, W), lambda b, r: (b, 2, r, 0)),
        ],
        out_specs=(
            pl.BlockSpec((1, ROWS, W), lambda b, r: (b, r, 0)),
            pl.BlockSpec((1, ROWS, W), lambda b, r: (b, r, 0)),
        ),
    )(flow_in, x)
    return lin.reshape(B * HW), pvn.reshape(B * HW)


def _bin_body(lin_hbm, pvn_hbm, pack_hbm, pvnb_hbm, cnt_hbm,
              lin_v, pvn_v, next_v, pstage, vstage, posb, sem):
    wid = lax.axis_index("s") * 2 + lax.axis_index("c")
    iota = lax.iota(jnp.int32, 16)

    def per_pc(k, _):
        pc = wid * 2 + k
        pt0 = pc * PCPTS
        p_base = (pc % PCB) * PCPTS

        def zero(i, _):
            next_v[pl.ds(i * 16, 16)] = jnp.zeros((16,), jnp.int32)
            return 0

        lax.fori_loop(0, 1024 // 16, zero, 0)

        def per_chunk(ch, _):
            off = pt0 + ch * ACH
            pltpu.sync_copy(lin_hbm.at[pl.ds(off, ACH)], lin_v)
            pltpu.sync_copy(pvn_hbm.at[pl.ds(off, ACH)], pvn_v)

            def vloop(i, _):
                l = lin_v[pl.ds(i * 16, 16)]
                v = pvn_v[pl.ds(i * 16, 16)]
                act = v > 0.0
                rk = l & 511
                key = jnp.where(act, rk, 1023)
                p = p_base + ch * ACH + i * 16 + iota
                pack = lax.shift_left(lax.shift_right_logical(l, 9), 18) | p
                skey, sval = plsc.sort_key_val(key, iota)
                v_s = jnp.take(v, sval)
                pack_s = jnp.take(pack, sval)
                act_s = skey < NBKT
                st = iota
                c = ((skey == jnp.take(skey, jnp.maximum(iota - 1, 0)))
                     & (iota >= 1)).astype(jnp.int32)
                for d in (1, 2, 4, 8):
                    back = jnp.maximum(iota - d, 0)
                    st = jnp.where(c != 0, jnp.take(st, back), st)
                    c = c & jnp.take(c, back)
                rank = iota - st
                nxt_key = jnp.take(skey, jnp.minimum(iota + 1, 15))
                is_last = (iota == 15) | (nxt_key != skey)
                cur = plsc.load_gather(next_v, [skey])
                pos = cur + rank
                valid = act_s & (pos < CAP)
                plsc.store_scatter(next_v, [skey], pos + 1, mask=is_last & act_s)
                gpos = (pc * NBKT + skey) * CAP + pos
                dump = DUMP + pt0 + ch * ACH + i * 16 + iota
                posb[pl.ds(i * 16, 16)] = jnp.where(valid, gpos, dump)
                pstage[pl.ds(i * 16, 16)] = pack_s
                vstage[pl.ds(i * 16, 16)] = v_s
                return 0

            lax.fori_loop(0, ACH // 16, vloop, 0)
            pltpu.async_copy(pstage, pack_hbm.at[posb], sem).wait()
            pltpu.async_copy(vstage, pvnb_hbm.at[posb], sem).wait()
            return 0

        lax.fori_loop(0, PCPTS // ACH, per_chunk, 0)
        pltpu.sync_copy(next_v.at[pl.ds(0, NBKT)],
                        cnt_hbm.at[pl.ds(pc * NBKT, NBKT)])
        return 0

    lax.fori_loop(0, 2, per_pc, 0)


def _splat_body(pack_hbm, pvnb_hbm, cnt_hbm, x_hbm, lin_hbm, pvn_hbm, out_hbm,
                pkbuf, pvbuf, cntv, maxv_fb, argp_fb, idx_v, gath_v, outb_v,
                lin_v, pvn_v, sem):
    wid = lax.axis_index("s") * 2 + lax.axis_index("c")
    base = wid * SHARD
    iota = lax.iota(jnp.int32, 16)

    def per_batch(b, _):
        pt_base = b * HW

        def init(i, _):
            maxv_fb[pl.ds(i * 16, 16)] = jnp.zeros((16,), jnp.float32)
            argp_fb[pl.ds(i * 16, 16)] = jnp.full((16,), HW, jnp.int32)
            return 0

        lax.fori_loop(0, SHARD // 16, init, 0)

        handles = []
        for j in range(PCB):
            pc = b * PCB + j
            boff = (pc * NBKT + wid * 16) * CAP
            handles.append(pltpu.async_copy(
                cnt_hbm.at[pl.ds(pc * NBKT + wid * 16, 16)],
                cntv.at[pl.ds(j * 16, 16)], sem))
            handles.append(pltpu.async_copy(
                pack_hbm.at[pl.ds(boff, 16 * CAP)],
                pkbuf.at[pl.ds(j * 16 * CAP, 16 * CAP)], sem))
            handles.append(pltpu.async_copy(
                pvnb_hbm.at[pl.ds(boff, 16 * CAP)],
                pvbuf.at[pl.ds(j * 16 * CAP, 16 * CAP)], sem))
        for h in handles:
            h.wait()

        def ovf_scan(j, m):
            cj = cntv[pl.ds(j * 16, 16)]
            return jnp.maximum(m, jnp.max(cj))

        max_cnt = lax.fori_loop(0, PCB, ovf_scan, jnp.int32(0))

        def pass1_j(j, _):
            cj = jnp.minimum(cntv[pl.ds(j * 16, 16)], CAP)
            trip = jnp.max(cj)
            bj = j * 16 * CAP

            def it(i, _):
                vidx = bj + iota * CAP + i
                pk = plsc.load_gather(pkbuf, [vidx])
                v = plsc.load_gather(pvbuf, [vidx])
                actm = i < cj
                low9 = lax.shift_right_logical(pk, 18)
                floc = iota * 512 + low9
                g = plsc.load_gather(maxv_fb, [floc])
                m = actm & (v > g)
                plsc.store_scatter(maxv_fb, [floc], v, mask=m)
                return 0

            lax.fori_loop(0, trip, it, 0)
            return 0

        lax.fori_loop(0, PCB, pass1_j, 0)

        @pl.when(max_cnt > CAP)
        def _():
            def f1_chunk(ci, _):
                off = pt_base + ci * ACH
                pltpu.sync_copy(lin_hbm.at[pl.ds(off, ACH)], lin_v)
                pltpu.sync_copy(pvn_hbm.at[pl.ds(off, ACH)], pvn_v)

                def vloop(i, _):
                    l = lin_v[pl.ds(i * 16, 16)]
                    v = pvn_v[pl.ds(i * 16, 16)]
                    cs = (l & 511) - wid * 16
                    act = (cs >= 0) & (cs < 16) & (v > 0.0)
                    locs = jnp.clip(cs, 0, 15) * 512 + lax.shift_right_logical(l, 9)
                    g = plsc.load_gather(maxv_fb, [locs])
                    need = act & (v > g)

                    def body(m):
                        plsc.store_scatter(maxv_fb, [locs], v, mask=m != 0)
                        g2 = plsc.load_gather(maxv_fb, [locs])
                        return (act & (v > g2)).astype(jnp.int32)

                    lax.while_loop(lambda m: jnp.any(m != 0), body,
                                   need.astype(jnp.int32))
                    return 0

                lax.fori_loop(0, ACH // 16, vloop, 0)
                return 0

            lax.fori_loop(0, HW // ACH, f1_chunk, 0)

        def pass2_j(j, _):
            cj = jnp.minimum(cntv[pl.ds(j * 16, 16)], CAP)
            trip = jnp.max(cj)
            bj = j * 16 * CAP

            def it(i, _):
                vidx = bj + iota * CAP + i
                pk = plsc.load_gather(pkbuf, [vidx])
                v = plsc.load_gather(pvbuf, [vidx])
                actm = i < cj
                low9 = lax.shift_right_logical(pk, 18)
                p = pk & 0x3FFFF
                floc = iota * 512 + low9
                g = plsc.load_gather(maxv_fb, [floc])
                win = actm & (v == g)
                ga = plsc.load_gather(argp_fb, [floc])
                m = win & (p < ga)
                plsc.store_scatter(argp_fb, [floc], p, mask=m)
                return 0

            lax.fori_loop(0, trip, it, 0)
            return 0

        lax.fori_loop(0, PCB, pass2_j, 0)

        @pl.when(max_cnt > CAP)
        def _():
            def f2_chunk(ci, _):
                off = pt_base + ci * ACH
                pltpu.sync_copy(lin_hbm.at[pl.ds(off, ACH)], lin_v)
                pltpu.sync_copy(pvn_hbm.at[pl.ds(off, ACH)], pvn_v)

                def vloop(i, _):
                    l = lin_v[pl.ds(i * 16, 16)]
                    v = pvn_v[pl.ds(i * 16, 16)]
                    cs = (l & 511) - wid * 16
                    act = (cs >= 0) & (cs < 16) & (v > 0.0)
                    locs = jnp.clip(cs, 0, 15) * 512 + lax.shift_right_logical(l, 9)
                    p = ci * ACH + i * 16 + iota
                    g = plsc.load_gather(maxv_fb, [locs])
                    win = act & (v == g)
                    ga = plsc.load_gather(argp_fb, [locs])
                    need = win & (p < ga)

                    def body(m):
                        plsc.store_scatter(argp_fb, [locs], p, mask=m != 0)
                        ga2 = plsc.load_gather(argp_fb, [locs])
                        return (win & (p < ga2)).astype(jnp.int32)

                    lax.while_loop(lambda m: jnp.any(m != 0), body,
                                   need.astype(jnp.int32))
                    return 0

                lax.fori_loop(0, ACH // 16, vloop, 0)
                return 0

            lax.fori_loop(0, HW // ACH, f2_chunk, 0)

        def mkidx(i, _):
            s = i * 16 + iota
            q = (s & 511) * 512 + wid * 16 + lax.shift_right_logical(s, 9)
            a = argp_fb[pl.ds(i * 16, 16)]
            valid = (a < HW) & (q > 0)
            idx_v[pl.ds(i * 16, 16)] = jnp.where(valid, a, q) + (b * C) * HW
            return 0

        lax.fori_loop(0, SHARD // 16, mkidx, 0)

        for c in range(C):
            if c > 0:
                def bump(i, _):
                    idx_v[pl.ds(i * 16, 16)] = idx_v[pl.ds(i * 16, 16)] + HW
                    return 0

                lax.fori_loop(0, SHARD // 16, bump, 0)
            pltpu.async_copy(x_hbm.at[idx_v], gath_v, sem).wait()

            def emit(i, _):
                s = i * 16 + iota
                y = s & 511
                sl = lax.shift_right_logical(s, 9)
                q = y * 512 + wid * 16 + sl
                a = argp_fb[pl.ds(i * 16, 16)]
                valid = (a < HW) & (q > 0)
                gv = gath_v[pl.ds(i * 16, 16)]
                val = jnp.where(valid & (gv < 10000.0), gv, 0.0)
                plsc.store_scatter(outb_v, [y * 16 + sl], val)
                return 0

            lax.fori_loop(0, SHARD // 16, emit, 0)
            pltpu.sync_copy(
                outb_v,
                out_hbm.at[pl.ds((wid * B * C + b * C + c) * SHARD, SHARD)])
        return 0

    lax.fori_loop(0, B, per_batch, 0)


@jax.jit
def kernel(x, flow_in):
    lin, pvn = _prep(x, flow_in)
    xf = x.reshape(B * C * HW)
    mesh = plsc.VectorSubcoreMesh(core_axis_name="c", subcore_axis_name="s")
    bink = functools.partial(
        pl.kernel,
        mesh=mesh,
        compiler_params=pltpu.CompilerParams(needs_layout_passes=False),
        out_type=(
            jax.ShapeDtypeStruct((BINSZ + B * HW,), jnp.int32),
            jax.ShapeDtypeStruct((BINSZ + B * HW,), jnp.float32),
            jax.ShapeDtypeStruct((NPC * NBKT,), jnp.int32),
        ),
        scratch_types=[
            pltpu.VMEM((ACH,), jnp.int32),
            pltpu.VMEM((ACH,), jnp.float32),
            pltpu.VMEM((1024,), jnp.int32),
            pltpu.VMEM((ACH,), jnp.int32),
            pltpu.VMEM((ACH,), jnp.float32),
            pltpu.VMEM((ACH,), jnp.int32),
            pltpu.SemaphoreType.DMA,
        ],
    )(_bin_body)
    pack_b, pvn_b, counts = bink(lin, pvn)

    splat = functools.partial(
        pl.kernel,
        mesh=mesh,
        compiler_params=pltpu.CompilerParams(needs_layout_passes=False),
        out_type=jax.ShapeDtypeStruct((NW * B * C * SHARD,), jnp.float32),
        scratch_types=[
            pltpu.VMEM((PCB * 16 * CAP,), jnp.int32),
            pltpu.VMEM((PCB * 16 * CAP,), jnp.float32),
            pltpu.VMEM((PCB * 16,), jnp.int32),
            pltpu.VMEM((SHARD,), jnp.float32),
            pltpu.VMEM((SHARD,), jnp.int32),
            pltpu.VMEM((SHARD,), jnp.int32),
            pltpu.VMEM((SHARD,), jnp.float32),
            pltpu.VMEM((SHARD,), jnp.float32),
            pltpu.VMEM((ACH,), jnp.int32),
            pltpu.VMEM((ACH,), jnp.float32),
            pltpu.SemaphoreType.DMA,
        ],
    )(_splat_body)
    out = splat(pack_b, pvn_b, counts, xf, lin, pvn)
    return out.reshape(NW, B, C, H, 16).transpose(1, 2, 3, 0, 4).reshape(B, C, H, W)
